# baseline (speedup 1.0000x reference)
import functools

import jax
import jax.numpy as jnp
from jax import lax
from jax.experimental import pallas as pl
from jax.experimental.pallas import tpu as pltpu
from jax.experimental.pallas import tpu_sc as plsc

HIDDEN = 64
FIELD_A = 1000
BATCH = 16384

NUM_CORES = 2
NUM_SUBCORES = 16
NUM_WORKERS = NUM_CORES * NUM_SUBCORES
B_PER_W = BATCH // NUM_WORKERS
CHUNK = 128
N_CHUNKS = B_PER_W // CHUNK
LANES = 16

_mesh = plsc.VectorSubcoreMesh(core_axis_name="c", subcore_axis_name="s")


@functools.partial(
    pl.kernel,
    mesh=_mesh,
    out_type=jax.ShapeDtypeStruct((BATCH, HIDDEN), jnp.float32),
    compiler_params=pltpu.CompilerParams(use_tc_tiling_on_sc=False),
    scratch_types=[
        pltpu.VMEM((B_PER_W,), jnp.int32),
        pltpu.VMEM((B_PER_W,), jnp.int32),
        pltpu.VMEM((N_CHUNKS, CHUNK), jnp.int32),
        pltpu.VMEM((B_PER_W, HIDDEN), jnp.float32),
        pltpu.SemaphoreType.DMA,
    ],
)
def _sc_gather(idx_a_hbm, idx_b_hbm, table_hbm, out_hbm,
               a_v, b_v, idx_v, rows_v, sem):
    wid = lax.axis_index("s") * NUM_CORES + lax.axis_index("c")
    base = wid * B_PER_W

    pltpu.sync_copy(idx_a_hbm.at[pl.ds(base, B_PER_W)], a_v)
    pltpu.sync_copy(idx_b_hbm.at[pl.ds(base, B_PER_W)], b_v)

    for j in range(N_CHUNKS):
        for k in range(CHUNK // LANES):
            s = pl.ds(j * CHUNK + k * LANES, LANES)
            idx_v[j, pl.ds(k * LANES, LANES)] = a_v[s] + b_v[s] * FIELD_A

    copies = []
    for j in range(N_CHUNKS):
        cp = pltpu.make_async_copy(
            table_hbm.at[idx_v.at[j]],
            rows_v.at[pl.ds(j * CHUNK, CHUNK)],
            sem,
        )
        cp.start()
        copies.append(cp)
    for cp in copies:
        cp.wait()

    pltpu.sync_copy(rows_v, out_hbm.at[pl.ds(base, B_PER_W)])


def kernel(idx_a, idx_b, table):
    return _sc_gather(idx_a, idx_b, table)


# speedup vs baseline: 1.0280x; 1.0280x over previous
import functools

import jax
import jax.numpy as jnp
from jax import lax
from jax.experimental import pallas as pl
from jax.experimental.pallas import tpu as pltpu
from jax.experimental.pallas import tpu_sc as plsc

HIDDEN = 64
FIELD_A = 1000
BATCH = 16384

NUM_CORES = 2
NUM_SUBCORES = 16
NUM_WORKERS = NUM_CORES * NUM_SUBCORES
B_PER_W = BATCH // NUM_WORKERS
LANES = 16

_mesh = plsc.VectorSubcoreMesh(core_axis_name="c", subcore_axis_name="s")


@functools.partial(
    pl.kernel,
    mesh=_mesh,
    out_type=jax.ShapeDtypeStruct((BATCH, HIDDEN), jnp.float32),
    scratch_types=[
        pltpu.VMEM((B_PER_W,), jnp.int32),
        pltpu.VMEM((B_PER_W,), jnp.int32),
        pltpu.SemaphoreType.DMA,
    ],
)
def _sc_gather(idx_a_hbm, idx_b_hbm, table_hbm, out_hbm, a_v, b_v, sem):
    wid = lax.axis_index("s") * NUM_CORES + lax.axis_index("c")
    base = wid * B_PER_W

    pltpu.sync_copy(idx_a_hbm.at[pl.ds(base, B_PER_W)], a_v)
    pltpu.sync_copy(idx_b_hbm.at[pl.ds(base, B_PER_W)], b_v)

    def fire(g):
        ev = a_v[pl.ds(g * LANES, LANES)] + b_v[pl.ds(g * LANES, LANES)] * FIELD_A
        for l in range(LANES):
            e = ev[l]
            pltpu.make_async_copy(
                table_hbm.at[pl.ds(e, 1)],
                out_hbm.at[pl.ds(base + g * LANES + l, 1)],
                sem,
            ).start()

    pl.loop(0, B_PER_W // LANES)(fire)

    pltpu.make_async_copy(
        table_hbm.at[pl.ds(0, B_PER_W)],
        out_hbm.at[pl.ds(base, B_PER_W)],
        sem,
    ).wait()


def kernel(idx_a, idx_b, table):
    return _sc_gather(idx_a, idx_b, table)


# speedup vs baseline: 1.7184x; 1.6716x over previous
import functools

import jax
import jax.numpy as jnp
from jax import lax
from jax.experimental import pallas as pl
from jax.experimental.pallas import tpu as pltpu
from jax.experimental.pallas import tpu_sc as plsc

HIDDEN = 64
FIELD_A = 1000
BATCH = 16384

NUM_CORES = 2
NUM_SUBCORES = 16
NUM_WORKERS = NUM_CORES * NUM_SUBCORES
B_PER_W = BATCH // NUM_WORKERS
LANES = 16

_mesh = plsc.VectorSubcoreMesh(core_axis_name="c", subcore_axis_name="s")


@functools.partial(
    pl.kernel,
    mesh=_mesh,
    out_type=jax.ShapeDtypeStruct((BATCH, HIDDEN), jnp.float32),
    scratch_types=[
        pltpu.VMEM((B_PER_W,), jnp.int32),
        pltpu.VMEM((B_PER_W,), jnp.int32),
        pltpu.VMEM((B_PER_W, HIDDEN), jnp.float32),
        pltpu.SemaphoreType.DMA,
    ],
)
def _sc_gather(idx_a_hbm, idx_b_hbm, table_hbm, out_hbm, a_v, b_v, rows_v, sem):
    wid = lax.axis_index("s") * NUM_CORES + lax.axis_index("c")
    base = wid * B_PER_W

    pltpu.sync_copy(idx_a_hbm.at[pl.ds(base, B_PER_W)], a_v)
    pltpu.sync_copy(idx_b_hbm.at[pl.ds(base, B_PER_W)], b_v)

    def fire(g):
        ev = a_v[pl.ds(g * LANES, LANES)] + b_v[pl.ds(g * LANES, LANES)] * FIELD_A
        for l in range(LANES):
            e = ev[l]
            pltpu.make_async_copy(
                table_hbm.at[pl.ds(e, 1)],
                rows_v.at[pl.ds(g * LANES + l, 1)],
                sem,
            ).start()

    pl.loop(0, B_PER_W // LANES)(fire)

    pltpu.make_async_copy(table_hbm.at[pl.ds(0, B_PER_W)], rows_v, sem).wait()

    pltpu.sync_copy(rows_v, out_hbm.at[pl.ds(base, B_PER_W)])


def kernel(idx_a, idx_b, table):
    return _sc_gather(idx_a, idx_b, table)
